# baseline (speedup 1.0000x reference)
import functools

import jax
import jax.numpy as jnp
from jax import lax
from jax.experimental import pallas as pl
from jax.experimental.pallas import tpu as pltpu
from jax.experimental.pallas import tpu_sc as plsc

N = 10000
E = 320000
D = 128
FD = 2 * D
NM = 3000
ROWS = 3072
NW = 32
EPW = E // NW
B = 80
SEG = 2000
NSEG = EPW // SEG
CAP = SEG + B
SOFF = ROWS * FD
SPAD = 4096
ACC = SOFF + SPAD


def _sc_accumulate(x, src, dst, w, comp, zeros):
    mesh = plsc.VectorSubcoreMesh(core_axis_name="c", subcore_axis_name="s")

    @functools.partial(
        pl.kernel,
        mesh=mesh,
        out_type=[
            jax.ShapeDtypeStruct((2, ROWS * FD), jnp.float32),
            jax.ShapeDtypeStruct((2, ROWS), jnp.float32),
        ],
        scratch_types=[
            pltpu.VMEM((N,), jnp.int32),
            pltpu.VMEM((SEG,), jnp.int32),
            pltpu.VMEM((SEG,), jnp.int32),
            pltpu.VMEM((SEG,), jnp.float32),
            pltpu.VMEM((CAP,), jnp.int32),
            pltpu.VMEM((CAP,), jnp.float32),
            pltpu.VMEM((CAP,), jnp.int32),
            pltpu.VMEM((CAP,), jnp.int32),
            pltpu.VMEM((B,), jnp.int32),
            pltpu.VMEM((B,), jnp.float32),
            pltpu.VMEM((B,), jnp.int32),
            pltpu.VMEM((B,), jnp.int32),
            pltpu.VMEM((B, D), jnp.float32),
            pltpu.VMEM((B * D + B,), jnp.float32),
            pltpu.VMEM((B * D + B,), jnp.int32),
            pltpu.VMEM_SHARED((ACC,), jnp.float32),
            pltpu.SemaphoreType.DMA,
        ],
        compiler_params=pltpu.CompilerParams(needs_layout_passes=False),
    )
    def body(x_h, src_h, dst_h, w_h, comp_h, zeros_h, out_h, outs_h,
             comp_v, ssb_v, dsb_v, wsb_v,
             stg_s, stg_w, stg_c, stg_m,
             sb_v, wb_v, csb_v, cidx_v, rows_v, tbuf_v, eidx_v, acc_s, sem):
        cid = lax.axis_index("c")
        sid = lax.axis_index("s")
        wid = sid * 2 + cid

        pltpu.sync_copy(comp_h, comp_v)

        spw = ACC // 16
        pltpu.sync_copy(zeros_h.at[pl.ds(sid * spw, spw)],
                        acc_s.at[pl.ds(sid * spw, spw)])
        plsc.subcore_barrier()

        lane = lax.iota(jnp.int32, 16)

        def segment(seg, carry):
            base = wid * EPW + seg * SEG
            pltpu.sync_copy(src_h.at[pl.ds(base, SEG)], ssb_v)
            pltpu.sync_copy(dst_h.at[pl.ds(base, SEG)], dsb_v)
            pltpu.sync_copy(w_h.at[pl.ds(base, SEG)], wsb_v)

            def scan(q, ncnt):
                sl = pl.ds(q * 16, 16)
                sch = ssb_v[sl]
                cd = plsc.load_gather(comp_v, [dsb_v[sl]])
                keep = cd < NM
                ki = keep.astype(jnp.int32)
                pc = plsc.cumsum(ki)
                pos = jnp.full((16,), ncnt, jnp.int32) + pc - ki
                plsc.store_scatter(stg_s, [pos], sch, mask=keep)
                plsc.store_scatter(stg_w, [pos], wsb_v[sl], mask=keep)
                plsc.store_scatter(stg_c, [pos], cd, mask=keep)
                plsc.store_scatter(stg_m, [pos],
                                   plsc.load_gather(comp_v, [sch]), mask=keep)
                return ncnt + jnp.max(pc)

            ncnt = lax.fori_loop(0, SEG // 16, scan, jnp.int32(0))

            for q in range(B // 16):
                sl = pl.ds(ncnt + q * 16, 16)
                stg_s[sl] = jnp.zeros((16,), jnp.int32)
                stg_w[sl] = jnp.zeros((16,), jnp.float32)
                stg_c[sl] = jnp.full((16,), NM, jnp.int32)
                stg_m[sl] = jnp.full((16,), NM, jnp.int32)

            nb2 = (ncnt + (B - 1)) // B

            def batch(it, c1):
                b2 = it * B
                for q in range(B // 16):
                    sl = pl.ds(q * 16, 16)
                    dsl = pl.ds(b2 + q * 16, 16)
                    sb_v[sl] = stg_s[dsl]
                    wb_v[sl] = stg_w[dsl]
                    cidx_v[sl] = stg_c[dsl]
                    csb_v[sl] = stg_m[dsl]

                pltpu.async_copy(x_h.at[sb_v], rows_v, sem).wait()

                def edge(r, c2):
                    rv = jnp.full((16,), r, jnp.int32)
                    wv = plsc.load_gather(wb_v, [rv])
                    mb = plsc.load_gather(csb_v, [rv]) < NM
                    cv = (plsc.load_gather(cidx_v, [rv]) * FD + lane
                          + mb.astype(jnp.int32) * D)
                    rbase = r * D
                    for j in range(D // 16):
                        tbuf_v[pl.ds(rbase + j * 16, 16)] = (
                            rows_v[r, pl.ds(j * 16, 16)] * wv)
                        eidx_v[pl.ds(rbase + j * 16, 16)] = cv + (j * 16)
                    return c2

                lax.fori_loop(0, B, edge, 0)

                for q in range(B // 16):
                    sl = pl.ds(q * 16, 16)
                    tl = pl.ds(B * D + q * 16, 16)
                    mb16 = csb_v[sl] < NM
                    tbuf_v[tl] = jnp.where(mb16, wb_v[sl],
                                           jnp.zeros((16,), jnp.float32))
                    eidx_v[tl] = cidx_v[sl] + SOFF

                pltpu.sync_copy(tbuf_v, acc_s.at[eidx_v], add=True)
                return c1

            lax.fori_loop(0, nb2, batch, 0)
            return carry

        lax.fori_loop(0, NSEG, segment, 0)
        plsc.subcore_barrier()

        spw_r = ROWS * FD // 16
        pltpu.sync_copy(acc_s.at[pl.ds(sid * spw_r, spw_r)],
                        out_h.at[cid, pl.ds(sid * spw_r, spw_r)])

        @pl.when(sid < ROWS // 256)
        def _():
            pltpu.sync_copy(acc_s.at[pl.ds(SOFF + sid * 256, 256)],
                            outs_h.at[cid, pl.ds(sid * 256, 256)])

    return body(x, src, dst, w, comp, zeros)


def _tc_tail(partials, spartials, mask_token, W_enc, b_enc, W_pred, b_pred):
    def body(p_ref, ps_ref, mt_ref, we_ref, be_ref, wp_ref, bp_ref, out_ref):
        s2 = p_ref[0] + p_ref[1]
        a = s2[:NM, :D]
        b = s2[:NM, D:]
        sv = ps_ref[0, :NM] + ps_ref[1, :NM]
        aggT = a + b
        aggC = a + sv * mt_ref[...]
        we = we_ref[...]
        be = be_ref[...]
        tea = jnp.maximum(
            jax.lax.dot(aggT, we, precision=jax.lax.Precision.HIGHEST) + be, 0.0)
        ctx = jnp.maximum(
            jax.lax.dot(aggC, we, precision=jax.lax.Precision.HIGHEST) + be, 0.0)
        pred = jax.lax.dot(ctx, wp_ref[...],
                           precision=jax.lax.Precision.HIGHEST) + bp_ref[...]
        d = pred - tea
        out_ref[...] = (jnp.sum(d * d) / (NM * D)).reshape(1, 1)

    return pl.pallas_call(
        body,
        out_shape=jax.ShapeDtypeStruct((1, 1), jnp.float32),
    )(partials, spartials, mask_token, W_enc, b_enc, W_pred, b_pred)


def kernel(x, edge_index, edge_weight, mask_token, W_enc, b_enc, W_pred, b_pred):
    perm = jax.random.permutation(jax.random.key(42), N)
    mask_idx = perm[:NM]
    comp = jnp.full((N,), NM, jnp.int32).at[mask_idx].set(
        jnp.arange(NM, dtype=jnp.int32))
    zeros = jnp.zeros((ACC,), jnp.float32)
    partials, spartials = _sc_accumulate(
        x, edge_index[0], edge_index[1], edge_weight, comp, zeros)
    loss = _tc_tail(partials.reshape(2, ROWS, FD),
                    spartials.reshape(2, ROWS, 1), mask_token,
                    W_enc, b_enc.reshape(1, D), W_pred, b_pred.reshape(1, D))
    return loss[0, 0]


# speedup vs baseline: 1.5042x; 1.5042x over previous
import functools

import jax
import jax.numpy as jnp
from jax import lax
from jax.experimental import pallas as pl
from jax.experimental.pallas import tpu as pltpu
from jax.experimental.pallas import tpu_sc as plsc

N = 10000
E = 320000
D = 128
FD = 2 * D
NM = 3000
ROWS = 3072
NW = 32
EPW = E // NW
B = 64
SEG = 2000
NSEG = EPW // SEG
CAP = SEG + B
SOFF = ROWS * FD
SPAD = 4096
ACC = SOFF + SPAD


def _sc_accumulate(x, src, dst, w, comp, zeros):
    mesh = plsc.VectorSubcoreMesh(core_axis_name="c", subcore_axis_name="s")

    @functools.partial(
        pl.kernel,
        mesh=mesh,
        out_type=[
            jax.ShapeDtypeStruct((2, ROWS * FD), jnp.float32),
            jax.ShapeDtypeStruct((2, ROWS), jnp.float32),
        ],
        scratch_types=[
            pltpu.VMEM((N,), jnp.int32),
            pltpu.VMEM((SEG,), jnp.int32),
            pltpu.VMEM((SEG,), jnp.int32),
            pltpu.VMEM((SEG,), jnp.float32),
            pltpu.VMEM((CAP,), jnp.int32),
            pltpu.VMEM((CAP,), jnp.float32),
            pltpu.VMEM((CAP,), jnp.int32),
            pltpu.VMEM((CAP,), jnp.int32),
            pltpu.VMEM((B, D), jnp.float32),
            pltpu.VMEM((B, D), jnp.float32),
            pltpu.VMEM((B * D + B,), jnp.float32),
            pltpu.VMEM((B * D + B,), jnp.float32),
            pltpu.VMEM((B * D + B,), jnp.int32),
            pltpu.VMEM((B * D + B,), jnp.int32),
            pltpu.VMEM_SHARED((ACC,), jnp.float32),
            pltpu.SemaphoreType.DMA,
            pltpu.SemaphoreType.DMA,
            pltpu.SemaphoreType.DMA,
            pltpu.SemaphoreType.DMA,
        ],
        compiler_params=pltpu.CompilerParams(needs_layout_passes=False),
    )
    def body(x_h, src_h, dst_h, w_h, comp_h, zeros_h, out_h, outs_h,
             comp_v, ssb_v, dsb_v, wsb_v,
             stg_s, stg_w, stg_c, stg_m,
             rows0, rows1, tbuf0, tbuf1, eidx0, eidx1, acc_s,
             gsem0, gsem1, ssem0, ssem1):
        cid = lax.axis_index("c")
        sid = lax.axis_index("s")
        wid = sid * 2 + cid

        pltpu.sync_copy(comp_h, comp_v)

        spw = ACC // 16
        pltpu.sync_copy(zeros_h.at[pl.ds(sid * spw, spw)],
                        acc_s.at[pl.ds(sid * spw, spw)])
        plsc.subcore_barrier()

        lane = lax.iota(jnp.int32, 16)

        bufs = ((rows0, tbuf0, eidx0, gsem0, ssem0),
                (rows1, tbuf1, eidx1, gsem1, ssem1))

        def _gather_start(b2, b):
            pltpu.async_copy(
                x_h.at[stg_s.at[pl.ds(b2, B)]], bufs[b][0], bufs[b][3])

        def _gather_wait(b):
            pltpu.make_async_copy(
                x_h.at[stg_s.at[pl.ds(0, B)]], bufs[b][0], bufs[b][3]).wait()

        def _scatter_start(b):
            pltpu.async_copy(
                bufs[b][1], acc_s.at[bufs[b][2]], bufs[b][4], add=True)

        def _scatter_wait(b):
            pltpu.make_async_copy(
                bufs[b][1], acc_s.at[bufs[b][2]], bufs[b][4]).wait()

        def segment(seg, carry):
            base = wid * EPW + seg * SEG
            pltpu.sync_copy(src_h.at[pl.ds(base, SEG)], ssb_v)
            pltpu.sync_copy(dst_h.at[pl.ds(base, SEG)], dsb_v)
            pltpu.sync_copy(w_h.at[pl.ds(base, SEG)], wsb_v)

            def scan(q, ncnt):
                sl = pl.ds(q * 16, 16)
                sch = ssb_v[sl]
                cd = plsc.load_gather(comp_v, [dsb_v[sl]])
                keep = cd < NM
                ki = keep.astype(jnp.int32)
                pc = plsc.cumsum(ki)
                pos = jnp.full((16,), ncnt, jnp.int32) + pc - ki
                plsc.store_scatter(stg_s, [pos], sch, mask=keep)
                plsc.store_scatter(stg_w, [pos], wsb_v[sl], mask=keep)
                plsc.store_scatter(stg_c, [pos], cd, mask=keep)
                plsc.store_scatter(stg_m, [pos],
                                   plsc.load_gather(comp_v, [sch]), mask=keep)
                return ncnt + jnp.max(pc)

            ncnt = lax.fori_loop(0, SEG // 16, scan, jnp.int32(0))

            for q in range(B // 16):
                sl = pl.ds(ncnt + q * 16, 16)
                stg_s[sl] = jnp.zeros((16,), jnp.int32)
                stg_w[sl] = jnp.zeros((16,), jnp.float32)
                stg_c[sl] = jnp.full((16,), NM, jnp.int32)
                stg_m[sl] = jnp.full((16,), NM, jnp.int32)

            nb2 = jnp.maximum((ncnt + (B - 1)) // B, 1)

            _gather_start(0, 0)

            def do_batch(it, b):
                b2 = it * B

                _gather_wait(b)

                @pl.when(it + 1 < nb2)
                def _():
                    _gather_start(b2 + B, 1 - b)

                @pl.when(it >= 2)
                def _():
                    _scatter_wait(b)

                rows_b, tbuf_b, eidx_b = bufs[b][0], bufs[b][1], bufs[b][2]

                def edge(r, c2):
                    rv = jnp.full((16,), b2 + r, jnp.int32)
                    wv = plsc.load_gather(stg_w, [rv])
                    mb = plsc.load_gather(stg_m, [rv]) < NM
                    cv = (plsc.load_gather(stg_c, [rv]) * FD + lane
                          + mb.astype(jnp.int32) * D)
                    rbase = r * D
                    for j in range(D // 16):
                        tbuf_b[pl.ds(rbase + j * 16, 16)] = (
                            rows_b[r, pl.ds(j * 16, 16)] * wv)
                        eidx_b[pl.ds(rbase + j * 16, 16)] = cv + (j * 16)
                    return c2

                lax.fori_loop(0, B, edge, 0)

                for q in range(B // 16):
                    sl = pl.ds(b2 + q * 16, 16)
                    tl = pl.ds(B * D + q * 16, 16)
                    mb16 = stg_m[sl] < NM
                    tbuf_b[tl] = jnp.where(mb16, stg_w[sl],
                                           jnp.zeros((16,), jnp.float32))
                    eidx_b[tl] = stg_c[sl] + SOFF

                _scatter_start(b)

            def pair(it2, c1):
                for b in range(2):
                    it = it2 * 2 + b

                    @pl.when(it < nb2)
                    def _():
                        do_batch(it, b)
                return c1

            lax.fori_loop(0, (nb2 + 1) // 2, pair, 0)

            _scatter_wait(0)

            @pl.when(nb2 >= 2)
            def _():
                _scatter_wait(1)
            return carry

        lax.fori_loop(0, NSEG, segment, 0)
        plsc.subcore_barrier()

        spw_r = ROWS * FD // 16
        pltpu.sync_copy(acc_s.at[pl.ds(sid * spw_r, spw_r)],
                        out_h.at[cid, pl.ds(sid * spw_r, spw_r)])

        @pl.when(sid < ROWS // 256)
        def _():
            pltpu.sync_copy(acc_s.at[pl.ds(SOFF + sid * 256, 256)],
                            outs_h.at[cid, pl.ds(sid * 256, 256)])

    return body(x, src, dst, w, comp, zeros)


def _tc_tail(partials, spartials, mask_token, W_enc, b_enc, W_pred, b_pred):
    def body(p_ref, ps_ref, mt_ref, we_ref, be_ref, wp_ref, bp_ref, out_ref):
        s2 = p_ref[0] + p_ref[1]
        a = s2[:NM, :D]
        b = s2[:NM, D:]
        sv = ps_ref[0, :NM] + ps_ref[1, :NM]
        aggT = a + b
        aggC = a + sv * mt_ref[...]
        we = we_ref[...]
        be = be_ref[...]
        tea = jnp.maximum(
            jax.lax.dot(aggT, we, precision=jax.lax.Precision.HIGHEST) + be, 0.0)
        ctx = jnp.maximum(
            jax.lax.dot(aggC, we, precision=jax.lax.Precision.HIGHEST) + be, 0.0)
        pred = jax.lax.dot(ctx, wp_ref[...],
                           precision=jax.lax.Precision.HIGHEST) + bp_ref[...]
        d = pred - tea
        out_ref[...] = (jnp.sum(d * d) / (NM * D)).reshape(1, 1)

    return pl.pallas_call(
        body,
        out_shape=jax.ShapeDtypeStruct((1, 1), jnp.float32),
    )(partials, spartials, mask_token, W_enc, b_enc, W_pred, b_pred)


def kernel(x, edge_index, edge_weight, mask_token, W_enc, b_enc, W_pred, b_pred):
    perm = jax.random.permutation(jax.random.key(42), N)
    mask_idx = perm[:NM]
    comp = jnp.full((N,), NM, jnp.int32).at[mask_idx].set(
        jnp.arange(NM, dtype=jnp.int32))
    zeros = jnp.zeros((ACC,), jnp.float32)
    partials, spartials = _sc_accumulate(
        x, edge_index[0], edge_index[1], edge_weight, comp, zeros)
    loss = _tc_tail(partials.reshape(2, ROWS, FD),
                    spartials.reshape(2, ROWS, 1), mask_token,
                    W_enc, b_enc.reshape(1, D), W_pred, b_pred.reshape(1, D))
    return loss[0, 0]


# speedup vs baseline: 1.5217x; 1.0116x over previous
import functools

import jax
import jax.numpy as jnp
from jax import lax
from jax.experimental import pallas as pl
from jax.experimental.pallas import tpu as pltpu
from jax.experimental.pallas import tpu_sc as plsc

N = 10000
E = 320000
D = 128
FD = 2 * D
NM = 3000
ROWS = 3072
NW = 32
EPW = E // NW
B = 64
SEG = 2000
NSEG = EPW // SEG
CAP = SEG + B
SOFF = ROWS * FD
SPAD = 4096
ACC = SOFF + SPAD


def _sc_accumulate(x, src, dst, w, comp, zeros):
    mesh = plsc.VectorSubcoreMesh(core_axis_name="c", subcore_axis_name="s")

    @functools.partial(
        pl.kernel,
        mesh=mesh,
        out_type=[
            jax.ShapeDtypeStruct((2, ROWS * FD), jnp.float32),
            jax.ShapeDtypeStruct((2, ROWS), jnp.float32),
        ],
        scratch_types=[
            pltpu.VMEM((N,), jnp.int32),
            pltpu.VMEM((SEG,), jnp.int32),
            pltpu.VMEM((SEG,), jnp.int32),
            pltpu.VMEM((SEG,), jnp.int32),
            pltpu.VMEM((SEG,), jnp.int32),
            pltpu.VMEM((SEG,), jnp.float32),
            pltpu.VMEM((SEG,), jnp.float32),
            pltpu.VMEM((CAP,), jnp.int32),
            pltpu.VMEM((CAP,), jnp.float32),
            pltpu.VMEM((CAP,), jnp.int32),
            pltpu.VMEM((CAP,), jnp.int32),
            pltpu.VMEM((B, D), jnp.float32),
            pltpu.VMEM((B, D), jnp.float32),
            pltpu.VMEM((B * D + B,), jnp.float32),
            pltpu.VMEM((B * D + B,), jnp.float32),
            pltpu.VMEM((B * D + B,), jnp.int32),
            pltpu.VMEM((B * D + B,), jnp.int32),
            pltpu.VMEM_SHARED((ACC,), jnp.float32),
            pltpu.SemaphoreType.DMA,
            pltpu.SemaphoreType.DMA,
            pltpu.SemaphoreType.DMA,
            pltpu.SemaphoreType.DMA,
            pltpu.SemaphoreType.DMA,
            pltpu.SemaphoreType.DMA,
        ],
        compiler_params=pltpu.CompilerParams(needs_layout_passes=False),
    )
    def body(x_h, src_h, dst_h, w_h, comp_h, zeros_h, out_h, outs_h,
             comp_v, ssb0, ssb1, dsb0, dsb1, wsb0, wsb1,
             stg_s, stg_w, stg_c, stg_m,
             rows0, rows1, tbuf0, tbuf1, eidx0, eidx1, acc_s,
             gsem0, gsem1, ssem0, ssem1, psem0, psem1):
        cid = lax.axis_index("c")
        sid = lax.axis_index("s")
        wid = sid * 2 + cid

        pltpu.sync_copy(comp_h, comp_v)

        spw = ACC // 16
        pltpu.sync_copy(zeros_h.at[pl.ds(sid * spw, spw)],
                        acc_s.at[pl.ds(sid * spw, spw)])
        plsc.subcore_barrier()

        lane = lax.iota(jnp.int32, 16)

        bufs = ((rows0, tbuf0, eidx0, gsem0, ssem0),
                (rows1, tbuf1, eidx1, gsem1, ssem1))
        p1bufs = ((ssb0, dsb0, wsb0, psem0), (ssb1, dsb1, wsb1, psem1))

        def _p1_start(seg, b):
            base = wid * EPW + seg * SEG
            ssb_b, dsb_b, wsb_b, psem_b = p1bufs[b]
            pltpu.async_copy(src_h.at[pl.ds(base, SEG)], ssb_b, psem_b)
            pltpu.async_copy(dst_h.at[pl.ds(base, SEG)], dsb_b, psem_b)
            pltpu.async_copy(w_h.at[pl.ds(base, SEG)], wsb_b, psem_b)

        def _p1_wait(b):
            ssb_b, dsb_b, wsb_b, psem_b = p1bufs[b]
            pltpu.make_async_copy(src_h.at[pl.ds(0, SEG)], ssb_b,
                                  psem_b).wait()
            pltpu.make_async_copy(dst_h.at[pl.ds(0, SEG)], dsb_b,
                                  psem_b).wait()
            pltpu.make_async_copy(w_h.at[pl.ds(0, SEG)], wsb_b,
                                  psem_b).wait()

        def _gather_start(b2, b):
            pltpu.async_copy(
                x_h.at[stg_s.at[pl.ds(b2, B)]], bufs[b][0], bufs[b][3])

        def _gather_wait(b):
            pltpu.make_async_copy(
                x_h.at[stg_s.at[pl.ds(0, B)]], bufs[b][0], bufs[b][3]).wait()

        def _scatter_start(b):
            pltpu.async_copy(
                bufs[b][1], acc_s.at[bufs[b][2]], bufs[b][4], add=True)

        def _scatter_wait(b):
            pltpu.make_async_copy(
                bufs[b][1], acc_s.at[bufs[b][2]], bufs[b][4]).wait()

        def do_segment(seg, sb):
            ssb_v, dsb_v, wsb_v, _ = p1bufs[sb]
            _p1_wait(sb)

            @pl.when(seg + 1 < NSEG)
            def _():
                _p1_start(seg + 1, 1 - sb)

            def scan(q, ncnt):
                sl = pl.ds(q * 16, 16)
                sch = ssb_v[sl]
                cd = plsc.load_gather(comp_v, [dsb_v[sl]])
                keep = cd < NM
                ki = keep.astype(jnp.int32)
                pc = plsc.cumsum(ki)
                pos = jnp.full((16,), ncnt, jnp.int32) + pc - ki
                plsc.store_scatter(stg_s, [pos], sch, mask=keep)
                plsc.store_scatter(stg_w, [pos], wsb_v[sl], mask=keep)
                plsc.store_scatter(stg_c, [pos], cd, mask=keep)
                plsc.store_scatter(stg_m, [pos],
                                   plsc.load_gather(comp_v, [sch]), mask=keep)
                return ncnt + jnp.max(pc)

            ncnt = lax.fori_loop(0, SEG // 16, scan, jnp.int32(0))

            for q in range(B // 16):
                sl = pl.ds(ncnt + q * 16, 16)
                stg_s[sl] = jnp.zeros((16,), jnp.int32)
                stg_w[sl] = jnp.zeros((16,), jnp.float32)
                stg_c[sl] = jnp.full((16,), NM, jnp.int32)
                stg_m[sl] = jnp.full((16,), NM, jnp.int32)

            nb2 = jnp.maximum((ncnt + (B - 1)) // B, 1)

            _gather_start(0, 0)

            def do_batch(it, b):
                b2 = it * B

                _gather_wait(b)

                @pl.when(it + 1 < nb2)
                def _():
                    _gather_start(b2 + B, 1 - b)

                @pl.when(it >= 2)
                def _():
                    _scatter_wait(b)

                rows_b, tbuf_b, eidx_b = bufs[b][0], bufs[b][1], bufs[b][2]

                def edge(r, c2):
                    rv = jnp.full((16,), b2 + r, jnp.int32)
                    wv = plsc.load_gather(stg_w, [rv])
                    mb = plsc.load_gather(stg_m, [rv]) < NM
                    cv = (plsc.load_gather(stg_c, [rv]) * FD + lane
                          + mb.astype(jnp.int32) * D)
                    rbase = r * D
                    for j in range(D // 16):
                        tbuf_b[pl.ds(rbase + j * 16, 16)] = (
                            rows_b[r, pl.ds(j * 16, 16)] * wv)
                        eidx_b[pl.ds(rbase + j * 16, 16)] = cv + (j * 16)
                    return c2

                lax.fori_loop(0, B, edge, 0)

                for q in range(B // 16):
                    sl = pl.ds(b2 + q * 16, 16)
                    tl = pl.ds(B * D + q * 16, 16)
                    mb16 = stg_m[sl] < NM
                    tbuf_b[tl] = jnp.where(mb16, stg_w[sl],
                                           jnp.zeros((16,), jnp.float32))
                    eidx_b[tl] = stg_c[sl] + SOFF

                _scatter_start(b)

            def pair(it2, c1):
                for b in range(2):
                    it = it2 * 2 + b

                    @pl.when(it < nb2)
                    def _():
                        do_batch(it, b)
                return c1

            lax.fori_loop(0, (nb2 + 1) // 2, pair, 0)

            _scatter_wait(0)

            @pl.when(nb2 >= 2)
            def _():
                _scatter_wait(1)

        _p1_start(0, 0)

        def segpair(s2, c0):
            for sb in range(2):
                seg = s2 * 2 + sb

                @pl.when(seg < NSEG)
                def _():
                    do_segment(seg, sb)
            return c0

        lax.fori_loop(0, (NSEG + 1) // 2, segpair, 0)
        plsc.subcore_barrier()

        spw_r = ROWS * FD // 16
        pltpu.sync_copy(acc_s.at[pl.ds(sid * spw_r, spw_r)],
                        out_h.at[cid, pl.ds(sid * spw_r, spw_r)])

        @pl.when(sid < ROWS // 256)
        def _():
            pltpu.sync_copy(acc_s.at[pl.ds(SOFF + sid * 256, 256)],
                            outs_h.at[cid, pl.ds(sid * 256, 256)])

    return body(x, src, dst, w, comp, zeros)


def _tc_tail(partials, spartials, mask_token, W_enc, b_enc, W_pred, b_pred):
    def body(p_ref, ps_ref, mt_ref, we_ref, be_ref, wp_ref, bp_ref, out_ref):
        s2 = p_ref[0] + p_ref[1]
        a = s2[:NM, :D]
        b = s2[:NM, D:]
        sv = ps_ref[0, :NM] + ps_ref[1, :NM]
        aggT = a + b
        aggC = a + sv * mt_ref[...]
        we = we_ref[...]
        be = be_ref[...]
        tea = jnp.maximum(
            jax.lax.dot(aggT, we, precision=jax.lax.Precision.HIGHEST) + be, 0.0)
        ctx = jnp.maximum(
            jax.lax.dot(aggC, we, precision=jax.lax.Precision.HIGHEST) + be, 0.0)
        pred = jax.lax.dot(ctx, wp_ref[...],
                           precision=jax.lax.Precision.HIGHEST) + bp_ref[...]
        d = pred - tea
        out_ref[...] = (jnp.sum(d * d) / (NM * D)).reshape(1, 1)

    return pl.pallas_call(
        body,
        out_shape=jax.ShapeDtypeStruct((1, 1), jnp.float32),
    )(partials, spartials, mask_token, W_enc, b_enc, W_pred, b_pred)


def kernel(x, edge_index, edge_weight, mask_token, W_enc, b_enc, W_pred, b_pred):
    perm = jax.random.permutation(jax.random.key(42), N)
    mask_idx = perm[:NM]
    comp = jnp.full((N,), NM, jnp.int32).at[mask_idx].set(
        jnp.arange(NM, dtype=jnp.int32))
    zeros = jnp.zeros((ACC,), jnp.float32)
    partials, spartials = _sc_accumulate(
        x, edge_index[0], edge_index[1], edge_weight, comp, zeros)
    loss = _tc_tail(partials.reshape(2, ROWS, FD),
                    spartials.reshape(2, ROWS, 1), mask_token,
                    W_enc, b_enc.reshape(1, D), W_pred, b_pred.reshape(1, D))
    return loss[0, 0]


# speedup vs baseline: 1.5290x; 1.0048x over previous
import functools

import jax
import jax.numpy as jnp
from jax import lax
from jax.experimental import pallas as pl
from jax.experimental.pallas import tpu as pltpu
from jax.experimental.pallas import tpu_sc as plsc

N = 10000
E = 320000
D = 128
FD = 2 * D
NM = 3000
ROWS = 3072
NW = 32
EPW = E // NW
B = 64
SEG = 2000
NSEG = EPW // SEG
CAP = SEG + B
SOFF = ROWS * FD
SPAD = 4096
ACC = SOFF + SPAD


def _sc_accumulate(x, src, dst, w, comp):
    mesh = plsc.VectorSubcoreMesh(core_axis_name="c", subcore_axis_name="s")

    @functools.partial(
        pl.kernel,
        mesh=mesh,
        out_type=[
            jax.ShapeDtypeStruct((2, ROWS * FD), jnp.float32),
            jax.ShapeDtypeStruct((2, ROWS), jnp.float32),
        ],
        scratch_types=[
            pltpu.VMEM((N,), jnp.int32),
            pltpu.VMEM((SEG,), jnp.int32),
            pltpu.VMEM((SEG,), jnp.int32),
            pltpu.VMEM((SEG,), jnp.int32),
            pltpu.VMEM((SEG,), jnp.int32),
            pltpu.VMEM((SEG,), jnp.float32),
            pltpu.VMEM((SEG,), jnp.float32),
            pltpu.VMEM((CAP,), jnp.int32),
            pltpu.VMEM((CAP,), jnp.float32),
            pltpu.VMEM((CAP,), jnp.int32),
            pltpu.VMEM((CAP,), jnp.int32),
            pltpu.VMEM((B, D), jnp.float32),
            pltpu.VMEM((B, D), jnp.float32),
            pltpu.VMEM((B * D + B,), jnp.float32),
            pltpu.VMEM((B * D + B,), jnp.float32),
            pltpu.VMEM((B * D + B,), jnp.int32),
            pltpu.VMEM((B * D + B,), jnp.int32),
            pltpu.VMEM_SHARED((ACC,), jnp.float32),
            pltpu.SemaphoreType.DMA,
            pltpu.SemaphoreType.DMA,
            pltpu.SemaphoreType.DMA,
            pltpu.SemaphoreType.DMA,
            pltpu.SemaphoreType.DMA,
            pltpu.SemaphoreType.DMA,
        ],
        compiler_params=pltpu.CompilerParams(needs_layout_passes=False),
    )
    def body(x_h, src_h, dst_h, w_h, comp_h, out_h, outs_h,
             comp_v, ssb0, ssb1, dsb0, dsb1, wsb0, wsb1,
             stg_s, stg_w, stg_c, stg_m,
             rows0, rows1, tbuf0, tbuf1, eidx0, eidx1, acc_s,
             gsem0, gsem1, ssem0, ssem1, psem0, psem1):
        cid = lax.axis_index("c")
        sid = lax.axis_index("s")
        wid = sid * 2 + cid

        pltpu.sync_copy(comp_h, comp_v)

        spw = ACC // 16

        def zchunk(k, c):
            tbuf0[pl.ds(k * 16, 16)] = jnp.zeros((16,), jnp.float32)
            return c

        lax.fori_loop(0, 6400 // 16, zchunk, 0)
        zchunks = [6144] * 7 + [6400]
        zoff = 0
        for zc in zchunks:
            pltpu.async_copy(tbuf0.at[pl.ds(0, zc)],
                             acc_s.at[pl.ds(sid * spw + zoff, zc)], gsem0)
            zoff += zc
        zoff = 0
        for zc in zchunks:
            pltpu.make_async_copy(tbuf0.at[pl.ds(0, zc)],
                                  acc_s.at[pl.ds(sid * spw + zoff, zc)],
                                  gsem0).wait()
            zoff += zc
        plsc.subcore_barrier()

        lane = lax.iota(jnp.int32, 16)

        bufs = ((rows0, tbuf0, eidx0, gsem0, ssem0),
                (rows1, tbuf1, eidx1, gsem1, ssem1))
        p1bufs = ((ssb0, dsb0, wsb0, psem0), (ssb1, dsb1, wsb1, psem1))

        def _p1_start(seg, b):
            base = wid * EPW + seg * SEG
            ssb_b, dsb_b, wsb_b, psem_b = p1bufs[b]
            pltpu.async_copy(src_h.at[pl.ds(base, SEG)], ssb_b, psem_b)
            pltpu.async_copy(dst_h.at[pl.ds(base, SEG)], dsb_b, psem_b)
            pltpu.async_copy(w_h.at[pl.ds(base, SEG)], wsb_b, psem_b)

        def _p1_wait(b):
            ssb_b, dsb_b, wsb_b, psem_b = p1bufs[b]
            pltpu.make_async_copy(src_h.at[pl.ds(0, SEG)], ssb_b,
                                  psem_b).wait()
            pltpu.make_async_copy(dst_h.at[pl.ds(0, SEG)], dsb_b,
                                  psem_b).wait()
            pltpu.make_async_copy(w_h.at[pl.ds(0, SEG)], wsb_b,
                                  psem_b).wait()

        def _gather_start(b2, b):
            pltpu.async_copy(
                x_h.at[stg_s.at[pl.ds(b2, B)]], bufs[b][0], bufs[b][3])

        def _gather_wait(b):
            pltpu.make_async_copy(
                x_h.at[stg_s.at[pl.ds(0, B)]], bufs[b][0], bufs[b][3]).wait()

        def _scatter_start(b):
            pltpu.async_copy(
                bufs[b][1], acc_s.at[bufs[b][2]], bufs[b][4], add=True)

        def _scatter_wait(b):
            pltpu.make_async_copy(
                bufs[b][1], acc_s.at[bufs[b][2]], bufs[b][4]).wait()

        def do_segment(seg, sb):
            ssb_v, dsb_v, wsb_v, _ = p1bufs[sb]
            _p1_wait(sb)

            @pl.when(seg + 1 < NSEG)
            def _():
                _p1_start(seg + 1, 1 - sb)

            def scan(q, ncnt):
                sl = pl.ds(q * 16, 16)
                sch = ssb_v[sl]
                cd = plsc.load_gather(comp_v, [dsb_v[sl]])
                keep = cd < NM
                ki = keep.astype(jnp.int32)
                pc = plsc.cumsum(ki)
                pos = jnp.full((16,), ncnt, jnp.int32) + pc - ki
                plsc.store_scatter(stg_s, [pos], sch, mask=keep)
                plsc.store_scatter(stg_w, [pos], wsb_v[sl], mask=keep)
                plsc.store_scatter(stg_c, [pos], cd, mask=keep)
                plsc.store_scatter(stg_m, [pos],
                                   plsc.load_gather(comp_v, [sch]), mask=keep)
                return ncnt + jnp.max(pc)

            ncnt = lax.fori_loop(0, SEG // 16, scan, jnp.int32(0))

            for q in range(B // 16):
                sl = pl.ds(ncnt + q * 16, 16)
                stg_s[sl] = jnp.zeros((16,), jnp.int32)
                stg_w[sl] = jnp.zeros((16,), jnp.float32)
                stg_c[sl] = jnp.full((16,), NM, jnp.int32)
                stg_m[sl] = jnp.full((16,), NM, jnp.int32)

            nb2 = jnp.maximum((ncnt + (B - 1)) // B, 1)

            _gather_start(0, 0)

            def do_batch(it, b):
                b2 = it * B

                _gather_wait(b)

                @pl.when(it + 1 < nb2)
                def _():
                    _gather_start(b2 + B, 1 - b)

                @pl.when(it >= 2)
                def _():
                    _scatter_wait(b)

                rows_b, tbuf_b, eidx_b = bufs[b][0], bufs[b][1], bufs[b][2]

                def edge(r, c2):
                    rv = jnp.full((16,), b2 + r, jnp.int32)
                    wv = plsc.load_gather(stg_w, [rv])
                    mb = plsc.load_gather(stg_m, [rv]) < NM
                    cv = (plsc.load_gather(stg_c, [rv]) * FD + lane
                          + mb.astype(jnp.int32) * D)
                    rbase = r * D
                    for j in range(D // 16):
                        tbuf_b[pl.ds(rbase + j * 16, 16)] = (
                            rows_b[r, pl.ds(j * 16, 16)] * wv)
                        eidx_b[pl.ds(rbase + j * 16, 16)] = cv + (j * 16)
                    return c2

                lax.fori_loop(0, B, edge, 0)

                for q in range(B // 16):
                    sl = pl.ds(b2 + q * 16, 16)
                    tl = pl.ds(B * D + q * 16, 16)
                    mb16 = stg_m[sl] < NM
                    tbuf_b[tl] = jnp.where(mb16, stg_w[sl],
                                           jnp.zeros((16,), jnp.float32))
                    eidx_b[tl] = stg_c[sl] + SOFF

                _scatter_start(b)

            def pair(it2, c1):
                for b in range(2):
                    it = it2 * 2 + b

                    @pl.when(it < nb2)
                    def _():
                        do_batch(it, b)
                return c1

            lax.fori_loop(0, (nb2 + 1) // 2, pair, 0)

            _scatter_wait(0)

            @pl.when(nb2 >= 2)
            def _():
                _scatter_wait(1)

        _p1_start(0, 0)

        def segpair(s2, c0):
            for sb in range(2):
                seg = s2 * 2 + sb

                @pl.when(seg < NSEG)
                def _():
                    do_segment(seg, sb)
            return c0

        lax.fori_loop(0, (NSEG + 1) // 2, segpair, 0)
        plsc.subcore_barrier()

        spw_r = ROWS * FD // 16
        pltpu.sync_copy(acc_s.at[pl.ds(sid * spw_r, spw_r)],
                        out_h.at[cid, pl.ds(sid * spw_r, spw_r)])

        @pl.when(sid < ROWS // 256)
        def _():
            pltpu.sync_copy(acc_s.at[pl.ds(SOFF + sid * 256, 256)],
                            outs_h.at[cid, pl.ds(sid * 256, 256)])

    return body(x, src, dst, w, comp)


def _tc_tail(partials, spartials, mask_token, W_enc, b_enc, W_pred, b_pred):
    def body(p_ref, ps_ref, mt_ref, we_ref, be_ref, wp_ref, bp_ref, out_ref):
        s2 = p_ref[0] + p_ref[1]
        a = s2[:NM, :D]
        b = s2[:NM, D:]
        sv = ps_ref[0, :NM] + ps_ref[1, :NM]
        aggT = a + b
        aggC = a + sv * mt_ref[...]
        we = we_ref[...]
        be = be_ref[...]
        tea = jnp.maximum(
            jax.lax.dot(aggT, we, precision=jax.lax.Precision.HIGHEST) + be, 0.0)
        ctx = jnp.maximum(
            jax.lax.dot(aggC, we, precision=jax.lax.Precision.HIGHEST) + be, 0.0)
        pred = jax.lax.dot(ctx, wp_ref[...],
                           precision=jax.lax.Precision.HIGHEST) + bp_ref[...]
        d = pred - tea
        out_ref[...] = (jnp.sum(d * d) / (NM * D)).reshape(1, 1)

    return pl.pallas_call(
        body,
        out_shape=jax.ShapeDtypeStruct((1, 1), jnp.float32),
    )(partials, spartials, mask_token, W_enc, b_enc, W_pred, b_pred)


def kernel(x, edge_index, edge_weight, mask_token, W_enc, b_enc, W_pred, b_pred):
    perm = jax.random.permutation(jax.random.key(42), N)
    mask_idx = perm[:NM]
    comp = jnp.full((N,), NM, jnp.int32).at[mask_idx].set(
        jnp.arange(NM, dtype=jnp.int32))
    partials, spartials = _sc_accumulate(
        x, edge_index[0], edge_index[1], edge_weight, comp)
    loss = _tc_tail(partials.reshape(2, ROWS, FD),
                    spartials.reshape(2, ROWS, 1), mask_token,
                    W_enc, b_enc.reshape(1, D), W_pred, b_pred.reshape(1, D))
    return loss[0, 0]


# speedup vs baseline: 1.9258x; 1.2595x over previous
import functools

import jax
import jax.numpy as jnp
from jax import lax
from jax.experimental import pallas as pl
from jax.experimental.pallas import tpu as pltpu
from jax.experimental.pallas import tpu_sc as plsc

N = 10000
E = 320000
D = 128
FD = 2 * D
NM = 3000
ROWS = 3072
NW = 32
EPW = E // NW
B = 32
SEG = 2000
NSEG = EPW // SEG
CAP = SEG + B
SOFF = ROWS * FD
SPAD = 4096
ACC = SOFF + SPAD


def _sc_accumulate(x, src, dst, w, comp):
    mesh = plsc.VectorSubcoreMesh(core_axis_name="c", subcore_axis_name="s")

    @functools.partial(
        pl.kernel,
        mesh=mesh,
        out_type=[
            jax.ShapeDtypeStruct((2, ROWS * FD), jnp.float32),
            jax.ShapeDtypeStruct((2, ROWS), jnp.float32),
        ],
        scratch_types=[
            pltpu.VMEM((N,), jnp.int32),
            pltpu.VMEM((SEG,), jnp.int32),
            pltpu.VMEM((SEG,), jnp.int32),
            pltpu.VMEM((SEG,), jnp.int32),
            pltpu.VMEM((SEG,), jnp.int32),
            pltpu.VMEM((SEG,), jnp.float32),
            pltpu.VMEM((SEG,), jnp.float32),
            pltpu.VMEM((CAP,), jnp.int32),
            pltpu.VMEM((CAP,), jnp.float32),
            pltpu.VMEM((CAP,), jnp.int32),
            pltpu.VMEM((CAP,), jnp.int32),
            pltpu.VMEM((B, D), jnp.float32),
            pltpu.VMEM((B, D), jnp.float32),
            pltpu.VMEM((B * D + B,), jnp.float32),
            pltpu.VMEM((B * D + B,), jnp.float32),
            pltpu.VMEM((B * D + B,), jnp.int32),
            pltpu.VMEM((B * D + B,), jnp.int32),
            pltpu.VMEM_SHARED((ACC,), jnp.float32),
            pltpu.SemaphoreType.DMA,
            pltpu.SemaphoreType.DMA,
            pltpu.SemaphoreType.DMA,
            pltpu.SemaphoreType.DMA,
            pltpu.SemaphoreType.DMA,
            pltpu.SemaphoreType.DMA,
        ],
        compiler_params=pltpu.CompilerParams(needs_layout_passes=False),
    )
    def body(x_h, src_h, dst_h, w_h, comp_h, out_h, outs_h,
             comp_v, ssb0, ssb1, dsb0, dsb1, wsb0, wsb1,
             stg_s, stg_w, stg_c, stg_m,
             rows0, rows1, tbuf0, tbuf1, eidx0, eidx1, acc_s,
             gsem0, gsem1, ssem0, ssem1, psem0, psem1):
        cid = lax.axis_index("c")
        sid = lax.axis_index("s")
        wid = sid * 2 + cid

        pltpu.sync_copy(comp_h, comp_v)

        spw = ACC // 16

        def zchunk(k, c):
            tbuf0[pl.ds(k * 16, 16)] = jnp.zeros((16,), jnp.float32)
            return c

        lax.fori_loop(0, 4096 // 16, zchunk, 0)
        zchunks = [4096] * 12 + [256]
        zoff = 0
        for zc in zchunks:
            pltpu.async_copy(tbuf0.at[pl.ds(0, zc)],
                             acc_s.at[pl.ds(sid * spw + zoff, zc)], gsem0)
            zoff += zc
        zoff = 0
        for zc in zchunks:
            pltpu.make_async_copy(tbuf0.at[pl.ds(0, zc)],
                                  acc_s.at[pl.ds(sid * spw + zoff, zc)],
                                  gsem0).wait()
            zoff += zc
        plsc.subcore_barrier()

        lane = lax.iota(jnp.int32, 16)

        bufs = ((rows0, tbuf0, eidx0, gsem0, ssem0),
                (rows1, tbuf1, eidx1, gsem1, ssem1))
        p1bufs = ((ssb0, dsb0, wsb0, psem0), (ssb1, dsb1, wsb1, psem1))

        def _p1_start(seg, b):
            base = wid * EPW + seg * SEG
            ssb_b, dsb_b, wsb_b, psem_b = p1bufs[b]
            pltpu.async_copy(src_h.at[pl.ds(base, SEG)], ssb_b, psem_b)
            pltpu.async_copy(dst_h.at[pl.ds(base, SEG)], dsb_b, psem_b)
            pltpu.async_copy(w_h.at[pl.ds(base, SEG)], wsb_b, psem_b)

        def _p1_wait(b):
            ssb_b, dsb_b, wsb_b, psem_b = p1bufs[b]
            pltpu.make_async_copy(src_h.at[pl.ds(0, SEG)], ssb_b,
                                  psem_b).wait()
            pltpu.make_async_copy(dst_h.at[pl.ds(0, SEG)], dsb_b,
                                  psem_b).wait()
            pltpu.make_async_copy(w_h.at[pl.ds(0, SEG)], wsb_b,
                                  psem_b).wait()

        def _gather_start(b2, b):
            pltpu.async_copy(
                x_h.at[stg_s.at[pl.ds(b2, B)]], bufs[b][0], bufs[b][3])

        def _gather_wait(b):
            pltpu.make_async_copy(
                x_h.at[stg_s.at[pl.ds(0, B)]], bufs[b][0], bufs[b][3]).wait()

        def _scatter_start(b):
            pltpu.async_copy(
                bufs[b][1], acc_s.at[bufs[b][2]], bufs[b][4], add=True)

        def _scatter_wait(b):
            pltpu.make_async_copy(
                bufs[b][1], acc_s.at[bufs[b][2]], bufs[b][4]).wait()

        def do_segment(seg, sb):
            ssb_v, dsb_v, wsb_v, _ = p1bufs[sb]
            _p1_wait(sb)

            @pl.when(seg + 1 < NSEG)
            def _():
                _p1_start(seg + 1, 1 - sb)

            def scan(q, ncnt):
                sl = pl.ds(q * 16, 16)
                sch = ssb_v[sl]
                cd = plsc.load_gather(comp_v, [dsb_v[sl]])
                keep = cd < NM
                ki = keep.astype(jnp.int32)
                pc = plsc.cumsum(ki)
                pos = jnp.full((16,), ncnt, jnp.int32) + pc - ki
                plsc.store_scatter(stg_s, [pos], sch, mask=keep)
                plsc.store_scatter(stg_w, [pos], wsb_v[sl], mask=keep)
                plsc.store_scatter(stg_c, [pos], cd, mask=keep)
                plsc.store_scatter(stg_m, [pos],
                                   plsc.load_gather(comp_v, [sch]), mask=keep)
                return ncnt + jnp.max(pc)

            ncnt = lax.fori_loop(0, SEG // 16, scan, jnp.int32(0))

            for q in range(B // 16):
                sl = pl.ds(ncnt + q * 16, 16)
                stg_s[sl] = jnp.zeros((16,), jnp.int32)
                stg_w[sl] = jnp.zeros((16,), jnp.float32)
                stg_c[sl] = jnp.full((16,), NM, jnp.int32)
                stg_m[sl] = jnp.full((16,), NM, jnp.int32)

            nb2 = jnp.maximum((ncnt + (B - 1)) // B, 1)

            _gather_start(0, 0)

            def do_batch(it, b):
                b2 = it * B

                _gather_wait(b)

                @pl.when(it + 1 < nb2)
                def _():
                    _gather_start(b2 + B, 1 - b)

                @pl.when(it >= 2)
                def _():
                    _scatter_wait(b)

                rows_b, tbuf_b, eidx_b = bufs[b][0], bufs[b][1], bufs[b][2]

                def edge(r, c2):
                    rv = jnp.full((16,), b2 + r, jnp.int32)
                    wv = plsc.load_gather(stg_w, [rv])
                    mb = plsc.load_gather(stg_m, [rv]) < NM
                    cv = (plsc.load_gather(stg_c, [rv]) * FD + lane
                          + mb.astype(jnp.int32) * D)
                    rbase = r * D
                    for j in range(D // 16):
                        tbuf_b[pl.ds(rbase + j * 16, 16)] = (
                            rows_b[r, pl.ds(j * 16, 16)] * wv)
                        eidx_b[pl.ds(rbase + j * 16, 16)] = cv + (j * 16)
                    return c2

                lax.fori_loop(0, B, edge, 0)

                for q in range(B // 16):
                    sl = pl.ds(b2 + q * 16, 16)
                    tl = pl.ds(B * D + q * 16, 16)
                    mb16 = stg_m[sl] < NM
                    tbuf_b[tl] = jnp.where(mb16, stg_w[sl],
                                           jnp.zeros((16,), jnp.float32))
                    eidx_b[tl] = stg_c[sl] + SOFF

                _scatter_start(b)

            def pair(it2, c1):
                for b in range(2):
                    it = it2 * 2 + b

                    @pl.when(it < nb2)
                    def _():
                        do_batch(it, b)
                return c1

            lax.fori_loop(0, (nb2 + 1) // 2, pair, 0)

            _scatter_wait(0)

            @pl.when(nb2 >= 2)
            def _():
                _scatter_wait(1)

        _p1_start(0, 0)

        def segpair(s2, c0):
            for sb in range(2):
                seg = s2 * 2 + sb

                @pl.when(seg < NSEG)
                def _():
                    do_segment(seg, sb)
            return c0

        lax.fori_loop(0, (NSEG + 1) // 2, segpair, 0)
        plsc.subcore_barrier()

        spw_r = ROWS * FD // 16
        pltpu.sync_copy(acc_s.at[pl.ds(sid * spw_r, spw_r)],
                        out_h.at[cid, pl.ds(sid * spw_r, spw_r)])

        @pl.when(sid < ROWS // 256)
        def _():
            pltpu.sync_copy(acc_s.at[pl.ds(SOFF + sid * 256, 256)],
                            outs_h.at[cid, pl.ds(sid * 256, 256)])

    return body(x, src, dst, w, comp)


def _tc_tail(partials, spartials, mask_token, W_enc, b_enc, W_pred, b_pred):
    def body(p_ref, ps_ref, mt_ref, we_ref, be_ref, wp_ref, bp_ref, out_ref):
        s2 = p_ref[0] + p_ref[1]
        a = s2[:NM, :D]
        b = s2[:NM, D:]
        sv = ps_ref[0, :NM] + ps_ref[1, :NM]
        aggT = a + b
        aggC = a + sv * mt_ref[...]
        we = we_ref[...]
        be = be_ref[...]
        tea = jnp.maximum(
            jax.lax.dot(aggT, we, precision=jax.lax.Precision.HIGHEST) + be, 0.0)
        ctx = jnp.maximum(
            jax.lax.dot(aggC, we, precision=jax.lax.Precision.HIGHEST) + be, 0.0)
        pred = jax.lax.dot(ctx, wp_ref[...],
                           precision=jax.lax.Precision.HIGHEST) + bp_ref[...]
        d = pred - tea
        out_ref[...] = (jnp.sum(d * d) / (NM * D)).reshape(1, 1)

    return pl.pallas_call(
        body,
        out_shape=jax.ShapeDtypeStruct((1, 1), jnp.float32),
    )(partials, spartials, mask_token, W_enc, b_enc, W_pred, b_pred)


def kernel(x, edge_index, edge_weight, mask_token, W_enc, b_enc, W_pred, b_pred):
    perm = jax.random.permutation(jax.random.key(42), N)
    mask_idx = perm[:NM]
    comp = jnp.full((N,), NM, jnp.int32).at[mask_idx].set(
        jnp.arange(NM, dtype=jnp.int32))
    partials, spartials = _sc_accumulate(
        x, edge_index[0], edge_index[1], edge_weight, comp)
    loss = _tc_tail(partials.reshape(2, ROWS, FD),
                    spartials.reshape(2, ROWS, 1), mask_token,
                    W_enc, b_enc.reshape(1, D), W_pred, b_pred.reshape(1, D))
    return loss[0, 0]


# speedup vs baseline: 2.0225x; 1.0502x over previous
import functools

import jax
import jax.numpy as jnp
from jax import lax
from jax.experimental import pallas as pl
from jax.experimental.pallas import tpu as pltpu
from jax.experimental.pallas import tpu_sc as plsc

N = 10000
E = 320000
D = 128
FD = 2 * D
NM = 3000
ROWS = 3072
NW = 32
EPW = E // NW
B = 16
SEG = 2000
NSEG = EPW // SEG
CAP = SEG + B
SOFF = ROWS * FD
SPAD = 4096
ACC = SOFF + SPAD


def _sc_accumulate(x, src, dst, w, comp):
    mesh = plsc.VectorSubcoreMesh(core_axis_name="c", subcore_axis_name="s")

    @functools.partial(
        pl.kernel,
        mesh=mesh,
        out_type=[
            jax.ShapeDtypeStruct((2, ROWS * FD), jnp.float32),
            jax.ShapeDtypeStruct((2, ROWS), jnp.float32),
        ],
        scratch_types=[
            pltpu.VMEM((N,), jnp.int32),
            pltpu.VMEM((SEG,), jnp.int32),
            pltpu.VMEM((SEG,), jnp.int32),
            pltpu.VMEM((SEG,), jnp.int32),
            pltpu.VMEM((SEG,), jnp.int32),
            pltpu.VMEM((SEG,), jnp.float32),
            pltpu.VMEM((SEG,), jnp.float32),
            pltpu.VMEM((CAP,), jnp.int32),
            pltpu.VMEM((CAP,), jnp.float32),
            pltpu.VMEM((CAP,), jnp.int32),
            pltpu.VMEM((CAP,), jnp.int32),
            pltpu.VMEM((B, D), jnp.float32),
            pltpu.VMEM((B, D), jnp.float32),
            pltpu.VMEM((B * D + B,), jnp.float32),
            pltpu.VMEM((B * D + B,), jnp.float32),
            pltpu.VMEM((B * D + B,), jnp.int32),
            pltpu.VMEM((B * D + B,), jnp.int32),
            pltpu.VMEM_SHARED((ACC,), jnp.float32),
            pltpu.SemaphoreType.DMA,
            pltpu.SemaphoreType.DMA,
            pltpu.SemaphoreType.DMA,
            pltpu.SemaphoreType.DMA,
            pltpu.SemaphoreType.DMA,
            pltpu.SemaphoreType.DMA,
        ],
        compiler_params=pltpu.CompilerParams(needs_layout_passes=False),
    )
    def body(x_h, src_h, dst_h, w_h, comp_h, out_h, outs_h,
             comp_v, ssb0, ssb1, dsb0, dsb1, wsb0, wsb1,
             stg_s, stg_w, stg_c, stg_m,
             rows0, rows1, tbuf0, tbuf1, eidx0, eidx1, acc_s,
             gsem0, gsem1, ssem0, ssem1, psem0, psem1):
        cid = lax.axis_index("c")
        sid = lax.axis_index("s")
        wid = sid * 2 + cid

        pltpu.sync_copy(comp_h, comp_v)

        spw = ACC // 16

        def zchunk(k, c):
            tbuf0[pl.ds(k * 16, 16)] = jnp.zeros((16,), jnp.float32)
            return c

        lax.fori_loop(0, 2048 // 16, zchunk, 0)
        zchunks = [2048] * 24 + [256]
        zoff = 0
        for zc in zchunks:
            pltpu.async_copy(tbuf0.at[pl.ds(0, zc)],
                             acc_s.at[pl.ds(sid * spw + zoff, zc)], gsem0)
            zoff += zc
        zoff = 0
        for zc in zchunks:
            pltpu.make_async_copy(tbuf0.at[pl.ds(0, zc)],
                                  acc_s.at[pl.ds(sid * spw + zoff, zc)],
                                  gsem0).wait()
            zoff += zc
        plsc.subcore_barrier()

        lane = lax.iota(jnp.int32, 16)

        bufs = ((rows0, tbuf0, eidx0, gsem0, ssem0),
                (rows1, tbuf1, eidx1, gsem1, ssem1))
        p1bufs = ((ssb0, dsb0, wsb0, psem0), (ssb1, dsb1, wsb1, psem1))

        def _p1_start(seg, b):
            base = wid * EPW + seg * SEG
            ssb_b, dsb_b, wsb_b, psem_b = p1bufs[b]
            pltpu.async_copy(src_h.at[pl.ds(base, SEG)], ssb_b, psem_b)
            pltpu.async_copy(dst_h.at[pl.ds(base, SEG)], dsb_b, psem_b)
            pltpu.async_copy(w_h.at[pl.ds(base, SEG)], wsb_b, psem_b)

        def _p1_wait(b):
            ssb_b, dsb_b, wsb_b, psem_b = p1bufs[b]
            pltpu.make_async_copy(src_h.at[pl.ds(0, SEG)], ssb_b,
                                  psem_b).wait()
            pltpu.make_async_copy(dst_h.at[pl.ds(0, SEG)], dsb_b,
                                  psem_b).wait()
            pltpu.make_async_copy(w_h.at[pl.ds(0, SEG)], wsb_b,
                                  psem_b).wait()

        def _gather_start(b2, b):
            pltpu.async_copy(
                x_h.at[stg_s.at[pl.ds(b2, B)]], bufs[b][0], bufs[b][3])

        def _gather_wait(b):
            pltpu.make_async_copy(
                x_h.at[stg_s.at[pl.ds(0, B)]], bufs[b][0], bufs[b][3]).wait()

        def _scatter_start(b):
            pltpu.async_copy(
                bufs[b][1], acc_s.at[bufs[b][2]], bufs[b][4], add=True)

        def _scatter_wait(b):
            pltpu.make_async_copy(
                bufs[b][1], acc_s.at[bufs[b][2]], bufs[b][4]).wait()

        def do_segment(seg, sb):
            ssb_v, dsb_v, wsb_v, _ = p1bufs[sb]
            _p1_wait(sb)

            @pl.when(seg + 1 < NSEG)
            def _():
                _p1_start(seg + 1, 1 - sb)

            def scan(q, ncnt):
                sl = pl.ds(q * 16, 16)
                sch = ssb_v[sl]
                cd = plsc.load_gather(comp_v, [dsb_v[sl]])
                keep = cd < NM
                ki = keep.astype(jnp.int32)
                pc = plsc.cumsum(ki)
                pos = jnp.full((16,), ncnt, jnp.int32) + pc - ki
                plsc.store_scatter(stg_s, [pos], sch, mask=keep)
                plsc.store_scatter(stg_w, [pos], wsb_v[sl], mask=keep)
                plsc.store_scatter(stg_c, [pos], cd, mask=keep)
                plsc.store_scatter(stg_m, [pos],
                                   plsc.load_gather(comp_v, [sch]), mask=keep)
                return ncnt + jnp.max(pc)

            ncnt = lax.fori_loop(0, SEG // 16, scan, jnp.int32(0))

            for q in range(B // 16):
                sl = pl.ds(ncnt + q * 16, 16)
                stg_s[sl] = jnp.zeros((16,), jnp.int32)
                stg_w[sl] = jnp.zeros((16,), jnp.float32)
                stg_c[sl] = jnp.full((16,), NM, jnp.int32)
                stg_m[sl] = jnp.full((16,), NM, jnp.int32)

            nb2 = jnp.maximum((ncnt + (B - 1)) // B, 1)

            _gather_start(0, 0)

            def do_batch(it, b):
                b2 = it * B

                _gather_wait(b)

                @pl.when(it + 1 < nb2)
                def _():
                    _gather_start(b2 + B, 1 - b)

                @pl.when(it >= 2)
                def _():
                    _scatter_wait(b)

                rows_b, tbuf_b, eidx_b = bufs[b][0], bufs[b][1], bufs[b][2]

                def edge(r, c2):
                    rv = jnp.full((16,), b2 + r, jnp.int32)
                    wv = plsc.load_gather(stg_w, [rv])
                    mb = plsc.load_gather(stg_m, [rv]) < NM
                    cv = (plsc.load_gather(stg_c, [rv]) * FD + lane
                          + mb.astype(jnp.int32) * D)
                    rbase = r * D
                    for j in range(D // 16):
                        tbuf_b[pl.ds(rbase + j * 16, 16)] = (
                            rows_b[r, pl.ds(j * 16, 16)] * wv)
                        eidx_b[pl.ds(rbase + j * 16, 16)] = cv + (j * 16)
                    return c2

                lax.fori_loop(0, B, edge, 0)

                for q in range(B // 16):
                    sl = pl.ds(b2 + q * 16, 16)
                    tl = pl.ds(B * D + q * 16, 16)
                    mb16 = stg_m[sl] < NM
                    tbuf_b[tl] = jnp.where(mb16, stg_w[sl],
                                           jnp.zeros((16,), jnp.float32))
                    eidx_b[tl] = stg_c[sl] + SOFF

                _scatter_start(b)

            def pair(it2, c1):
                for b in range(2):
                    it = it2 * 2 + b

                    @pl.when(it < nb2)
                    def _():
                        do_batch(it, b)
                return c1

            lax.fori_loop(0, (nb2 + 1) // 2, pair, 0)

            _scatter_wait(0)

            @pl.when(nb2 >= 2)
            def _():
                _scatter_wait(1)

        _p1_start(0, 0)

        def segpair(s2, c0):
            for sb in range(2):
                seg = s2 * 2 + sb

                @pl.when(seg < NSEG)
                def _():
                    do_segment(seg, sb)
            return c0

        lax.fori_loop(0, (NSEG + 1) // 2, segpair, 0)
        plsc.subcore_barrier()

        spw_r = ROWS * FD // 16
        pltpu.sync_copy(acc_s.at[pl.ds(sid * spw_r, spw_r)],
                        out_h.at[cid, pl.ds(sid * spw_r, spw_r)])

        @pl.when(sid < ROWS // 256)
        def _():
            pltpu.sync_copy(acc_s.at[pl.ds(SOFF + sid * 256, 256)],
                            outs_h.at[cid, pl.ds(sid * 256, 256)])

    return body(x, src, dst, w, comp)


def _tc_tail(partials, spartials, mask_token, W_enc, b_enc, W_pred, b_pred):
    def body(p_ref, ps_ref, mt_ref, we_ref, be_ref, wp_ref, bp_ref, out_ref):
        s2 = p_ref[0] + p_ref[1]
        a = s2[:NM, :D]
        b = s2[:NM, D:]
        sv = ps_ref[0, :NM] + ps_ref[1, :NM]
        aggT = a + b
        aggC = a + sv * mt_ref[...]
        we = we_ref[...]
        be = be_ref[...]
        tea = jnp.maximum(
            jax.lax.dot(aggT, we, precision=jax.lax.Precision.HIGHEST) + be, 0.0)
        ctx = jnp.maximum(
            jax.lax.dot(aggC, we, precision=jax.lax.Precision.HIGHEST) + be, 0.0)
        pred = jax.lax.dot(ctx, wp_ref[...],
                           precision=jax.lax.Precision.HIGHEST) + bp_ref[...]
        d = pred - tea
        out_ref[...] = (jnp.sum(d * d) / (NM * D)).reshape(1, 1)

    return pl.pallas_call(
        body,
        out_shape=jax.ShapeDtypeStruct((1, 1), jnp.float32),
    )(partials, spartials, mask_token, W_enc, b_enc, W_pred, b_pred)


def kernel(x, edge_index, edge_weight, mask_token, W_enc, b_enc, W_pred, b_pred):
    perm = jax.random.permutation(jax.random.key(42), N)
    mask_idx = perm[:NM]
    comp = jnp.full((N,), NM, jnp.int32).at[mask_idx].set(
        jnp.arange(NM, dtype=jnp.int32))
    partials, spartials = _sc_accumulate(
        x, edge_index[0], edge_index[1], edge_weight, comp)
    loss = _tc_tail(partials.reshape(2, ROWS, FD),
                    spartials.reshape(2, ROWS, 1), mask_token,
                    W_enc, b_enc.reshape(1, D), W_pred, b_pred.reshape(1, D))
    return loss[0, 0]
